# baseline (speedup 1.0000x reference)
import functools

import jax
import jax.numpy as jnp
from jax import lax
from jax.experimental import pallas as pl
from jax.experimental.pallas import tpu as pltpu
from jax.experimental.pallas import tpu_sc as plsc

N = 10000
E = 320000
D = 128
NUM_NODE_TYPES = 64
R2 = 16
EPS = 0.01

BLK = 2000
NBLK = N // BLK

NW = 32
TPE = E // NW
KB = 128
NBF = TPE // KB
TAIL = TPE - NBF * KB
NPAD = 10240
RPT = NPAD // 16


def _rel_body(q_ref, xs_ref, rel_ref):
    Qr = q_ref[0]
    row = lax.broadcasted_iota(jnp.int32, (D, D), 0)
    col = lax.broadcasted_iota(jnp.int32, (D, D), 1)
    eye = jnp.where(row == col, 1.0, 0.0).astype(jnp.float32)
    W = eye + xs_ref[0]
    WQ = jnp.dot(W, Qr, preferred_element_type=jnp.float32)
    rel_ref[0] = lax.dot_general(Qr, WQ, (((0,), (0,)), ((), ())),
                                 preferred_element_type=jnp.float32)


def _rel_call(Qsel, Xs):
    return pl.pallas_call(
        _rel_body,
        grid=(R2,),
        in_specs=[
            pl.BlockSpec((1, D, D), lambda r: (r, 0, 0)),
            pl.BlockSpec((1, D, D), lambda r: (r, 0, 0)),
        ],
        out_specs=pl.BlockSpec((1, D, D), lambda r: (r, 0, 0)),
        out_shape=jax.ShapeDtypeStruct((R2, D, D), jnp.float32),
    )(Qsel, Xs)


def _h_body(idx_ref, nt_ref, rel_ref, h_ref, x_scr):
    r = pl.program_id(1)

    @pl.when(r == 0)
    def _():
        idx = idx_ref[0, 0, :]
        iota = lax.broadcasted_iota(jnp.int32, (BLK, NUM_NODE_TYPES), 1)
        hit = (idx[:, None] == iota) & (idx[:, None] >= 0)
        onehot = jnp.where(hit, 1.0, 0.0).astype(jnp.float32)
        x_scr[...] = jnp.dot(onehot, nt_ref[...],
                             preferred_element_type=jnp.float32)

    h_ref[0] = lax.dot_general(x_scr[...], rel_ref[0], (((1,), (1,)), ((), ())),
                               preferred_element_type=jnp.float32)


def _h_call(idx3, node_table, rel):
    return pl.pallas_call(
        _h_body,
        grid=(NBLK, R2),
        in_specs=[
            pl.BlockSpec((1, 1, BLK), lambda nb, r: (nb, 0, 0)),
            pl.BlockSpec((NUM_NODE_TYPES, D), lambda nb, r: (0, 0)),
            pl.BlockSpec((1, D, D), lambda nb, r: (r, 0, 0)),
        ],
        out_specs=pl.BlockSpec((1, BLK, D), lambda nb, r: (r, nb, 0)),
        out_shape=jax.ShapeDtypeStruct((R2, N, D), jnp.float32),
        scratch_shapes=[pltpu.VMEM((BLK, D), jnp.float32)],
    )(idx3, node_table, rel)


_MESH = plsc.VectorSubcoreMesh(core_axis_name="c", subcore_axis_name="s")

NBF = TPE // KB
TAIL = TPE - NBF * KB


@functools.partial(
    pl.kernel,
    mesh=_MESH,
    out_type=jax.ShapeDtypeStruct((2 * NPAD, D), jnp.float32),
    scratch_types=[
        pltpu.VMEM((KB,), jnp.int32),
        pltpu.VMEM((KB,), jnp.int32),
        pltpu.VMEM((KB,), jnp.int32),
        pltpu.VMEM((KB,), jnp.int32),
        pltpu.VMEM((KB,), jnp.int32),
        pltpu.VMEM((KB,), jnp.int32),
        pltpu.VMEM((KB,), jnp.int32),
        pltpu.VMEM((KB,), jnp.int32),
        pltpu.VMEM((KB,), jnp.int32),
        pltpu.VMEM((KB,), jnp.int32),
        pltpu.VMEM((KB, D), jnp.float32),
        pltpu.VMEM((KB, D), jnp.float32),
        pltpu.VMEM((TAIL,), jnp.int32),
        pltpu.VMEM((TAIL,), jnp.int32),
        pltpu.VMEM((TAIL,), jnp.int32),
        pltpu.VMEM((TAIL,), jnp.int32),
        pltpu.VMEM((TAIL, D), jnp.float32),
        pltpu.VMEM_SHARED((NPAD, D), jnp.float32),
        pltpu.SemaphoreType.DMA,
        pltpu.SemaphoreType.DMA,
        pltpu.SemaphoreType.DMA,
        pltpu.SemaphoreType.DMA,
        pltpu.SemaphoreType.DMA,
        pltpu.SemaphoreType.DMA,
    ],
)
def _sc_edges(h_hbm, src_hbm, typ_hbm, dst_hbm, out_hbm,
              srcA, srcB, typA, typB, dstA, dstB, gidxA, gidxB, sidxA, sidxB,
              rowsA, rowsB, srcT, typT, dstT, gidxT, rowsT,
              acc, semIA, semIB, semGA, semGB, semSA, semSB):
    c = lax.axis_index("c")
    s = lax.axis_index("s")
    wid = s * 2 + c

    zeros16 = jnp.zeros((16,), jnp.float32)

    def zrow(i, carry):
        for j in range(D // 16):
            rowsA[i, pl.ds(j * 16, 16)] = zeros16
        return carry

    lax.fori_loop(0, KB, zrow, 0)
    for cpy in range(RPT // KB):
        pltpu.sync_copy(rowsA, acc.at[pl.ds(s * RPT + cpy * KB, KB)])
    plsc.subcore_barrier()

    ebase = wid * TPE

    def fire_idx(i, srcv, typv, dstv, sem):
        off = ebase + i * KB
        pltpu.async_copy(src_hbm.at[pl.ds(off, KB)], srcv, sem)
        pltpu.async_copy(typ_hbm.at[pl.ds(off, KB)], typv, sem)
        pltpu.async_copy(dst_hbm.at[pl.ds(off, KB)], dstv, sem)

    def finish_idx(srcv, typv, dstv, gidx, sidx, sem):
        pltpu.make_async_copy(src_hbm.at[pl.ds(0, KB)], srcv, sem).wait()
        pltpu.make_async_copy(typ_hbm.at[pl.ds(0, KB)], typv, sem).wait()
        pltpu.make_async_copy(dst_hbm.at[pl.ds(0, KB)], dstv, sem).wait()
        for j in range(KB // 16):
            sl = pl.ds(j * 16, 16)
            gidx[sl] = typv[sl] * N + srcv[sl]
            sidx[sl] = dstv[sl]

    def fire_gather(gidx, rows, sem):
        pltpu.async_copy(h_hbm.at[gidx], rows, sem)

    def wait_gather(gidx, rows, sem):
        pltpu.make_async_copy(h_hbm.at[gidx], rows, sem).wait()

    def fire_scat(dstv, rows, sem):
        pltpu.async_copy(rows, acc.at[dstv], sem, add=True)

    def wait_scat(dstv, rows, sem):
        pltpu.make_async_copy(rows, acc.at[dstv], sem).wait()

    fire_idx(0, srcA, typA, dstA, semIA)
    fire_idx(1, srcB, typB, dstB, semIB)
    finish_idx(srcA, typA, dstA, gidxA, sidxA, semIA)
    fire_gather(gidxA, rowsA, semGA)
    fire_idx(2, srcA, typA, dstA, semIA)
    finish_idx(srcB, typB, dstB, gidxB, sidxB, semIB)
    fire_gather(gidxB, rowsB, semGB)
    fire_idx(3, srcB, typB, dstB, semIB)
    wait_gather(gidxA, rowsA, semGA)
    fire_scat(sidxA, rowsA, semSA)
    wait_gather(gidxB, rowsB, semGB)
    fire_scat(sidxB, rowsB, semSB)

    def pair(g, carry):
        wait_scat(sidxA, rowsA, semSA)
        finish_idx(srcA, typA, dstA, gidxA, sidxA, semIA)
        fire_gather(gidxA, rowsA, semGA)
        fire_idx(2 * g + 4, srcA, typA, dstA, semIA)
        wait_scat(sidxB, rowsB, semSB)
        finish_idx(srcB, typB, dstB, gidxB, sidxB, semIB)
        fire_gather(gidxB, rowsB, semGB)
        fire_idx(2 * g + 5, srcB, typB, dstB, semIB)
        wait_gather(gidxA, rowsA, semGA)
        fire_scat(sidxA, rowsA, semSA)
        wait_gather(gidxB, rowsB, semGB)
        fire_scat(sidxB, rowsB, semSB)
        return carry

    lax.fori_loop(0, NBF // 2 - 2, pair, 0)

    wait_scat(sidxA, rowsA, semSA)
    finish_idx(srcA, typA, dstA, gidxA, sidxA, semIA)
    fire_gather(gidxA, rowsA, semGA)
    wait_scat(sidxB, rowsB, semSB)
    finish_idx(srcB, typB, dstB, gidxB, sidxB, semIB)
    fire_gather(gidxB, rowsB, semGB)
    wait_gather(gidxA, rowsA, semGA)
    fire_scat(sidxA, rowsA, semSA)
    wait_gather(gidxB, rowsB, semGB)
    fire_scat(sidxB, rowsB, semSB)

    toff = ebase + NBF * KB
    pltpu.sync_copy(src_hbm.at[pl.ds(toff, TAIL)], srcT)
    pltpu.sync_copy(typ_hbm.at[pl.ds(toff, TAIL)], typT)
    pltpu.sync_copy(dst_hbm.at[pl.ds(toff, TAIL)], dstT)
    gidxT[...] = typT[...] * N + srcT[...]
    pltpu.async_copy(h_hbm.at[gidxT], rowsT, semGA).wait()
    pltpu.sync_copy(rowsT, acc.at[dstT], add=True)

    wait_scat(sidxA, rowsA, semSA)
    wait_scat(sidxB, rowsB, semSB)
    plsc.subcore_barrier()

    pltpu.sync_copy(acc.at[pl.ds(s * RPT, RPT)],
                    out_hbm.at[pl.ds(c * NPAD + s * RPT, RPT)])


def _add_body(p_ref, o_ref):
    o_ref[...] = p_ref[0] + p_ref[1]


def _add_call(partials):
    return pl.pallas_call(
        _add_body,
        grid=(NBLK,),
        in_specs=[pl.BlockSpec((2, BLK, D), lambda nb: (0, nb, 0))],
        out_specs=pl.BlockSpec((BLK, D), lambda nb: (nb, 0)),
        out_shape=jax.ShapeDtypeStruct((N, D), jnp.float32),
    )(partials)


def kernel(node_mapping, relation_mapping, edge_index, edge_type,
           node_table, rel_X, Q):
    Qsel = jnp.take(Q, relation_mapping[:, 0], axis=0)
    worder = relation_mapping[:, 1]
    sign = jnp.where(worder % 2 == 0, EPS, -EPS).astype(jnp.float32)
    Xs = jnp.take(rel_X, worder // 2, axis=0) * sign[:, None, None]

    rel = _rel_call(Qsel, Xs)

    idx3 = node_mapping[:, 1].astype(jnp.int32).reshape(NBLK, 1, BLK)
    h = _h_call(idx3, node_table, rel)
    h2 = h.reshape(R2 * N, D)

    src = edge_index[0].astype(jnp.int32)
    typ = edge_type.astype(jnp.int32)
    dst = edge_index[1].astype(jnp.int32)
    partials = _sc_edges(h2, src, typ, dst)
    return _add_call(partials.reshape(2, NPAD, D))


# speedup vs baseline: 1.0252x; 1.0252x over previous
import functools

import jax
import jax.numpy as jnp
from jax import lax
from jax.experimental import pallas as pl
from jax.experimental.pallas import tpu as pltpu
from jax.experimental.pallas import tpu_sc as plsc

N = 10000
E = 320000
D = 128
NUM_NODE_TYPES = 64
R2 = 16
EPS = 0.01

BLK = 2000
NBLK = N // BLK

NW = 32
TPE = E // NW
KB = 128
NBF = TPE // KB
TAIL = TPE - NBF * KB
NPAD = 10240
RPT = NPAD // 16


def _h_body(idx_ref, nt_ref, q_ref, xs_ref, h_ref, x_scr, rel_scr):
    nb = pl.program_id(0)
    r = pl.program_id(1)

    @pl.when(nb == 0)
    def _():
        Qr = q_ref[0]
        row = lax.broadcasted_iota(jnp.int32, (D, D), 0)
        col = lax.broadcasted_iota(jnp.int32, (D, D), 1)
        eye = jnp.where(row == col, 1.0, 0.0).astype(jnp.float32)
        W = eye + xs_ref[0]
        WQ = jnp.dot(W, Qr, preferred_element_type=jnp.float32)
        rel_scr[r, :, :] = lax.dot_general(Qr, WQ, (((0,), (0,)), ((), ())),
                                           preferred_element_type=jnp.float32)

    @pl.when(r == 0)
    def _():
        idx = idx_ref[0, 0, :]
        iota = lax.broadcasted_iota(jnp.int32, (BLK, NUM_NODE_TYPES), 1)
        hit = (idx[:, None] == iota) & (idx[:, None] >= 0)
        onehot = jnp.where(hit, 1.0, 0.0).astype(jnp.float32)
        x_scr[...] = jnp.dot(onehot, nt_ref[...],
                             preferred_element_type=jnp.float32)

    h_ref[0] = lax.dot_general(x_scr[...], rel_scr[r], (((1,), (1,)), ((), ())),
                               preferred_element_type=jnp.float32)


def _h_call(idx3, node_table, Qsel, Xs):
    return pl.pallas_call(
        _h_body,
        grid=(NBLK, R2),
        in_specs=[
            pl.BlockSpec((1, 1, BLK), lambda nb, r: (nb, 0, 0)),
            pl.BlockSpec((NUM_NODE_TYPES, D), lambda nb, r: (0, 0)),
            pl.BlockSpec((1, D, D), lambda nb, r: (r, 0, 0)),
            pl.BlockSpec((1, D, D), lambda nb, r: (r, 0, 0)),
        ],
        out_specs=pl.BlockSpec((1, BLK, D), lambda nb, r: (r, nb, 0)),
        out_shape=jax.ShapeDtypeStruct((R2, N, D), jnp.float32),
        scratch_shapes=[pltpu.VMEM((BLK, D), jnp.float32),
                        pltpu.VMEM((R2, D, D), jnp.float32)],
    )(idx3, node_table, Qsel, Xs)


_MESH = plsc.VectorSubcoreMesh(core_axis_name="c", subcore_axis_name="s")

NBF = TPE // KB
TAIL = TPE - NBF * KB


@functools.partial(
    pl.kernel,
    mesh=_MESH,
    out_type=jax.ShapeDtypeStruct((2 * NPAD, D), jnp.float32),
    scratch_types=[
        pltpu.VMEM((KB,), jnp.int32),
        pltpu.VMEM((KB,), jnp.int32),
        pltpu.VMEM((KB,), jnp.int32),
        pltpu.VMEM((KB,), jnp.int32),
        pltpu.VMEM((KB,), jnp.int32),
        pltpu.VMEM((KB,), jnp.int32),
        pltpu.VMEM((KB,), jnp.int32),
        pltpu.VMEM((KB,), jnp.int32),
        pltpu.VMEM((KB,), jnp.int32),
        pltpu.VMEM((KB,), jnp.int32),
        pltpu.VMEM((KB, D), jnp.float32),
        pltpu.VMEM((KB, D), jnp.float32),
        pltpu.VMEM((TAIL,), jnp.int32),
        pltpu.VMEM((TAIL,), jnp.int32),
        pltpu.VMEM((TAIL,), jnp.int32),
        pltpu.VMEM((TAIL,), jnp.int32),
        pltpu.VMEM((TAIL, D), jnp.float32),
        pltpu.VMEM_SHARED((NPAD, D), jnp.float32),
        pltpu.SemaphoreType.DMA,
        pltpu.SemaphoreType.DMA,
        pltpu.SemaphoreType.DMA,
        pltpu.SemaphoreType.DMA,
        pltpu.SemaphoreType.DMA,
        pltpu.SemaphoreType.DMA,
    ],
)
def _sc_edges(h_hbm, src_hbm, typ_hbm, dst_hbm, out_hbm,
              srcA, srcB, typA, typB, dstA, dstB, gidxA, gidxB, sidxA, sidxB,
              rowsA, rowsB, srcT, typT, dstT, gidxT, rowsT,
              acc, semIA, semIB, semGA, semGB, semSA, semSB):
    c = lax.axis_index("c")
    s = lax.axis_index("s")
    wid = s * 2 + c

    zeros16 = jnp.zeros((16,), jnp.float32)

    def zrow(i, carry):
        for j in range(D // 16):
            rowsA[i, pl.ds(j * 16, 16)] = zeros16
        return carry

    lax.fori_loop(0, KB, zrow, 0)
    for cpy in range(RPT // KB):
        pltpu.sync_copy(rowsA, acc.at[pl.ds(s * RPT + cpy * KB, KB)])
    plsc.subcore_barrier()

    ebase = wid * TPE

    def fire_idx(i, srcv, typv, dstv, sem):
        off = ebase + i * KB
        pltpu.async_copy(src_hbm.at[pl.ds(off, KB)], srcv, sem)
        pltpu.async_copy(typ_hbm.at[pl.ds(off, KB)], typv, sem)
        pltpu.async_copy(dst_hbm.at[pl.ds(off, KB)], dstv, sem)

    def finish_idx(srcv, typv, dstv, gidx, sidx, sem):
        pltpu.make_async_copy(src_hbm.at[pl.ds(0, KB)], srcv, sem).wait()
        pltpu.make_async_copy(typ_hbm.at[pl.ds(0, KB)], typv, sem).wait()
        pltpu.make_async_copy(dst_hbm.at[pl.ds(0, KB)], dstv, sem).wait()
        for j in range(KB // 16):
            sl = pl.ds(j * 16, 16)
            gidx[sl] = typv[sl] * N + srcv[sl]
            sidx[sl] = dstv[sl]

    def fire_gather(gidx, rows, sem):
        pltpu.async_copy(h_hbm.at[gidx], rows, sem)

    def wait_gather(gidx, rows, sem):
        pltpu.make_async_copy(h_hbm.at[gidx], rows, sem).wait()

    def fire_scat(dstv, rows, sem):
        pltpu.async_copy(rows, acc.at[dstv], sem, add=True)

    def wait_scat(dstv, rows, sem):
        pltpu.make_async_copy(rows, acc.at[dstv], sem).wait()

    fire_idx(0, srcA, typA, dstA, semIA)
    fire_idx(1, srcB, typB, dstB, semIB)
    finish_idx(srcA, typA, dstA, gidxA, sidxA, semIA)
    fire_gather(gidxA, rowsA, semGA)
    fire_idx(2, srcA, typA, dstA, semIA)
    finish_idx(srcB, typB, dstB, gidxB, sidxB, semIB)
    fire_gather(gidxB, rowsB, semGB)
    fire_idx(3, srcB, typB, dstB, semIB)
    wait_gather(gidxA, rowsA, semGA)
    fire_scat(sidxA, rowsA, semSA)
    wait_gather(gidxB, rowsB, semGB)
    fire_scat(sidxB, rowsB, semSB)

    def pair(g, carry):
        wait_scat(sidxA, rowsA, semSA)
        finish_idx(srcA, typA, dstA, gidxA, sidxA, semIA)
        fire_gather(gidxA, rowsA, semGA)
        fire_idx(2 * g + 4, srcA, typA, dstA, semIA)
        wait_scat(sidxB, rowsB, semSB)
        finish_idx(srcB, typB, dstB, gidxB, sidxB, semIB)
        fire_gather(gidxB, rowsB, semGB)
        fire_idx(2 * g + 5, srcB, typB, dstB, semIB)
        wait_gather(gidxA, rowsA, semGA)
        fire_scat(sidxA, rowsA, semSA)
        wait_gather(gidxB, rowsB, semGB)
        fire_scat(sidxB, rowsB, semSB)
        return carry

    lax.fori_loop(0, NBF // 2 - 2, pair, 0)

    wait_scat(sidxA, rowsA, semSA)
    finish_idx(srcA, typA, dstA, gidxA, sidxA, semIA)
    fire_gather(gidxA, rowsA, semGA)
    wait_scat(sidxB, rowsB, semSB)
    finish_idx(srcB, typB, dstB, gidxB, sidxB, semIB)
    fire_gather(gidxB, rowsB, semGB)
    wait_gather(gidxA, rowsA, semGA)
    fire_scat(sidxA, rowsA, semSA)
    wait_gather(gidxB, rowsB, semGB)
    fire_scat(sidxB, rowsB, semSB)

    toff = ebase + NBF * KB
    pltpu.sync_copy(src_hbm.at[pl.ds(toff, TAIL)], srcT)
    pltpu.sync_copy(typ_hbm.at[pl.ds(toff, TAIL)], typT)
    pltpu.sync_copy(dst_hbm.at[pl.ds(toff, TAIL)], dstT)
    gidxT[...] = typT[...] * N + srcT[...]
    pltpu.async_copy(h_hbm.at[gidxT], rowsT, semGA).wait()
    pltpu.sync_copy(rowsT, acc.at[dstT], add=True)

    wait_scat(sidxA, rowsA, semSA)
    wait_scat(sidxB, rowsB, semSB)
    plsc.subcore_barrier()

    pltpu.sync_copy(acc.at[pl.ds(s * RPT, RPT)],
                    out_hbm.at[pl.ds(c * NPAD + s * RPT, RPT)])


def _add_body(p_ref, o_ref):
    o_ref[...] = p_ref[0] + p_ref[1]


def _add_call(partials):
    return pl.pallas_call(
        _add_body,
        grid=(NBLK,),
        in_specs=[pl.BlockSpec((2, BLK, D), lambda nb: (0, nb, 0))],
        out_specs=pl.BlockSpec((BLK, D), lambda nb: (nb, 0)),
        out_shape=jax.ShapeDtypeStruct((N, D), jnp.float32),
    )(partials)


def kernel(node_mapping, relation_mapping, edge_index, edge_type,
           node_table, rel_X, Q):
    Qsel = jnp.take(Q, relation_mapping[:, 0], axis=0)
    worder = relation_mapping[:, 1]
    sign = jnp.where(worder % 2 == 0, EPS, -EPS).astype(jnp.float32)
    Xs = jnp.take(rel_X, worder // 2, axis=0) * sign[:, None, None]

    idx3 = node_mapping[:, 1].astype(jnp.int32).reshape(NBLK, 1, BLK)
    h = _h_call(idx3, node_table, Qsel, Xs)
    h2 = h.reshape(R2 * N, D)

    src = edge_index[0].astype(jnp.int32)
    typ = edge_type.astype(jnp.int32)
    dst = edge_index[1].astype(jnp.int32)
    partials = _sc_edges(h2, src, typ, dst)
    return _add_call(partials.reshape(2, NPAD, D))


# speedup vs baseline: 1.0278x; 1.0025x over previous
import functools

import jax
import jax.numpy as jnp
from jax import lax
from jax.experimental import pallas as pl
from jax.experimental.pallas import tpu as pltpu
from jax.experimental.pallas import tpu_sc as plsc

N = 10000
E = 320000
D = 128
NUM_NODE_TYPES = 64
R2 = 16
EPS = 0.01

BLK = 2000
NBLK = N // BLK

NW = 32
TPE = E // NW
KB = 128
NBF = TPE // KB
TAIL = TPE - NBF * KB
NPAD = 10240
RPT = NPAD // 16


def _h_body(idx_ref, nt_ref, q_ref, xs_ref, h_ref, x_scr, rel_scr):
    nb = pl.program_id(0)
    r = pl.program_id(1)

    @pl.when(nb == 0)
    def _():
        Qr = q_ref[0]
        row = lax.broadcasted_iota(jnp.int32, (D, D), 0)
        col = lax.broadcasted_iota(jnp.int32, (D, D), 1)
        eye = jnp.where(row == col, 1.0, 0.0).astype(jnp.float32)
        W = eye + xs_ref[0]
        WQ = jnp.dot(W, Qr, preferred_element_type=jnp.float32)
        rel_scr[r, :, :] = lax.dot_general(Qr, WQ, (((0,), (0,)), ((), ())),
                                           preferred_element_type=jnp.float32)

    @pl.when(r == 0)
    def _():
        idx = idx_ref[0, 0, :]
        iota = lax.broadcasted_iota(jnp.int32, (BLK, NUM_NODE_TYPES), 1)
        hit = (idx[:, None] == iota) & (idx[:, None] >= 0)
        onehot = jnp.where(hit, 1.0, 0.0).astype(jnp.float32)
        x_scr[...] = jnp.dot(onehot, nt_ref[...],
                             preferred_element_type=jnp.float32)

    h_ref[0] = lax.dot_general(x_scr[...], rel_scr[r], (((1,), (1,)), ((), ())),
                               preferred_element_type=jnp.float32)


def _h_call(idx3, node_table, Qsel, Xs):
    return pl.pallas_call(
        _h_body,
        grid=(NBLK, R2),
        in_specs=[
            pl.BlockSpec((1, 1, BLK), lambda nb, r: (nb, 0, 0)),
            pl.BlockSpec((NUM_NODE_TYPES, D), lambda nb, r: (0, 0)),
            pl.BlockSpec((1, D, D), lambda nb, r: (r, 0, 0)),
            pl.BlockSpec((1, D, D), lambda nb, r: (r, 0, 0)),
        ],
        out_specs=pl.BlockSpec((1, BLK, D), lambda nb, r: (r, nb, 0)),
        out_shape=jax.ShapeDtypeStruct((R2, N, D), jnp.float32),
        scratch_shapes=[pltpu.VMEM((BLK, D), jnp.float32),
                        pltpu.VMEM((R2, D, D), jnp.float32)],
    )(idx3, node_table, Qsel, Xs)


_MESH = plsc.VectorSubcoreMesh(core_axis_name="c", subcore_axis_name="s")

NBF = TPE // KB
TAIL = TPE - NBF * KB


@functools.partial(
    pl.kernel,
    mesh=_MESH,
    out_type=jax.ShapeDtypeStruct((2 * NPAD, D), jnp.float32),
    scratch_types=[
        pltpu.VMEM((KB,), jnp.int32),
        pltpu.VMEM((KB,), jnp.int32),
        pltpu.VMEM((KB,), jnp.int32),
        pltpu.VMEM((KB,), jnp.int32),
        pltpu.VMEM((KB,), jnp.int32),
        pltpu.VMEM((KB,), jnp.int32),
        pltpu.VMEM((KB,), jnp.int32),
        pltpu.VMEM((KB,), jnp.int32),
        pltpu.VMEM((KB,), jnp.int32),
        pltpu.VMEM((KB,), jnp.int32),
        pltpu.VMEM((KB, D), jnp.float32),
        pltpu.VMEM((KB, D), jnp.float32),
        pltpu.VMEM((TAIL,), jnp.int32),
        pltpu.VMEM((TAIL,), jnp.int32),
        pltpu.VMEM((TAIL,), jnp.int32),
        pltpu.VMEM((TAIL,), jnp.int32),
        pltpu.VMEM((TAIL, D), jnp.float32),
        pltpu.VMEM_SHARED((NPAD, D), jnp.float32),
        pltpu.SemaphoreType.DMA,
        pltpu.SemaphoreType.DMA,
        pltpu.SemaphoreType.DMA,
        pltpu.SemaphoreType.DMA,
        pltpu.SemaphoreType.DMA,
        pltpu.SemaphoreType.DMA,
    ],
)
def _sc_edges(h_hbm, src_hbm, typ_hbm, dst_hbm, out_hbm,
              srcA, srcB, typA, typB, dstA, dstB, gidxA, gidxB, sidxA, sidxB,
              rowsA, rowsB, srcT, typT, dstT, gidxT, rowsT,
              acc, semIA, semIB, semGA, semGB, semSA, semSB):
    c = lax.axis_index("c")
    s = lax.axis_index("s")
    wid = s * 2 + c

    ebase = wid * TPE

    def fire_idx(i, srcv, typv, dstv, sem):
        off = ebase + i * KB
        pltpu.async_copy(src_hbm.at[pl.ds(off, KB)], srcv, sem)
        pltpu.async_copy(typ_hbm.at[pl.ds(off, KB)], typv, sem)
        pltpu.async_copy(dst_hbm.at[pl.ds(off, KB)], dstv, sem)

    fire_idx(0, srcA, typA, dstA, semIA)
    fire_idx(1, srcB, typB, dstB, semIB)

    zeros16 = jnp.zeros((16,), jnp.float32)

    def zrow(i, carry):
        for j in range(D // 16):
            rowsA[i, pl.ds(j * 16, 16)] = zeros16
        return carry

    lax.fori_loop(0, KB, zrow, 0)
    for cpy in range(RPT // KB):
        pltpu.sync_copy(rowsA, acc.at[pl.ds(s * RPT + cpy * KB, KB)])
    plsc.subcore_barrier()

    def finish_idx(srcv, typv, dstv, gidx, sidx, sem):
        pltpu.make_async_copy(src_hbm.at[pl.ds(0, KB)], srcv, sem).wait()
        pltpu.make_async_copy(typ_hbm.at[pl.ds(0, KB)], typv, sem).wait()
        pltpu.make_async_copy(dst_hbm.at[pl.ds(0, KB)], dstv, sem).wait()
        for j in range(KB // 16):
            sl = pl.ds(j * 16, 16)
            gidx[sl] = typv[sl] * N + srcv[sl]
            sidx[sl] = dstv[sl]

    def fire_gather(gidx, rows, sem):
        pltpu.async_copy(h_hbm.at[gidx], rows, sem)

    def wait_gather(gidx, rows, sem):
        pltpu.make_async_copy(h_hbm.at[gidx], rows, sem).wait()

    def fire_scat(dstv, rows, sem):
        pltpu.async_copy(rows, acc.at[dstv], sem, add=True)

    def wait_scat(dstv, rows, sem):
        pltpu.make_async_copy(rows, acc.at[dstv], sem).wait()

    finish_idx(srcA, typA, dstA, gidxA, sidxA, semIA)
    fire_gather(gidxA, rowsA, semGA)
    fire_idx(2, srcA, typA, dstA, semIA)
    finish_idx(srcB, typB, dstB, gidxB, sidxB, semIB)
    fire_gather(gidxB, rowsB, semGB)
    fire_idx(3, srcB, typB, dstB, semIB)
    wait_gather(gidxA, rowsA, semGA)
    fire_scat(sidxA, rowsA, semSA)
    wait_gather(gidxB, rowsB, semGB)
    fire_scat(sidxB, rowsB, semSB)

    def pair(g, carry):
        wait_scat(sidxA, rowsA, semSA)
        finish_idx(srcA, typA, dstA, gidxA, sidxA, semIA)
        fire_gather(gidxA, rowsA, semGA)
        fire_idx(2 * g + 4, srcA, typA, dstA, semIA)
        wait_scat(sidxB, rowsB, semSB)
        finish_idx(srcB, typB, dstB, gidxB, sidxB, semIB)
        fire_gather(gidxB, rowsB, semGB)
        fire_idx(2 * g + 5, srcB, typB, dstB, semIB)
        wait_gather(gidxA, rowsA, semGA)
        fire_scat(sidxA, rowsA, semSA)
        wait_gather(gidxB, rowsB, semGB)
        fire_scat(sidxB, rowsB, semSB)
        return carry

    lax.fori_loop(0, NBF // 2 - 2, pair, 0)

    wait_scat(sidxA, rowsA, semSA)
    finish_idx(srcA, typA, dstA, gidxA, sidxA, semIA)
    fire_gather(gidxA, rowsA, semGA)
    wait_scat(sidxB, rowsB, semSB)
    finish_idx(srcB, typB, dstB, gidxB, sidxB, semIB)
    fire_gather(gidxB, rowsB, semGB)
    wait_gather(gidxA, rowsA, semGA)
    fire_scat(sidxA, rowsA, semSA)
    wait_gather(gidxB, rowsB, semGB)
    fire_scat(sidxB, rowsB, semSB)

    toff = ebase + NBF * KB
    pltpu.sync_copy(src_hbm.at[pl.ds(toff, TAIL)], srcT)
    pltpu.sync_copy(typ_hbm.at[pl.ds(toff, TAIL)], typT)
    pltpu.sync_copy(dst_hbm.at[pl.ds(toff, TAIL)], dstT)
    gidxT[...] = typT[...] * N + srcT[...]
    pltpu.async_copy(h_hbm.at[gidxT], rowsT, semGA).wait()
    pltpu.sync_copy(rowsT, acc.at[dstT], add=True)

    wait_scat(sidxA, rowsA, semSA)
    wait_scat(sidxB, rowsB, semSB)
    plsc.subcore_barrier()

    pltpu.sync_copy(acc.at[pl.ds(s * RPT, RPT)],
                    out_hbm.at[pl.ds(c * NPAD + s * RPT, RPT)])


def _add_body(p_ref, o_ref):
    o_ref[...] = p_ref[0] + p_ref[1]


def _add_call(partials):
    return pl.pallas_call(
        _add_body,
        grid=(NBLK,),
        in_specs=[pl.BlockSpec((2, BLK, D), lambda nb: (0, nb, 0))],
        out_specs=pl.BlockSpec((BLK, D), lambda nb: (nb, 0)),
        out_shape=jax.ShapeDtypeStruct((N, D), jnp.float32),
    )(partials)


def kernel(node_mapping, relation_mapping, edge_index, edge_type,
           node_table, rel_X, Q):
    Qsel = jnp.take(Q, relation_mapping[:, 0], axis=0)
    worder = relation_mapping[:, 1]
    sign = jnp.where(worder % 2 == 0, EPS, -EPS).astype(jnp.float32)
    Xs = jnp.take(rel_X, worder // 2, axis=0) * sign[:, None, None]

    idx3 = node_mapping[:, 1].astype(jnp.int32).reshape(NBLK, 1, BLK)
    h = _h_call(idx3, node_table, Qsel, Xs)
    h2 = h.reshape(R2 * N, D)

    src = edge_index[0].astype(jnp.int32)
    typ = edge_type.astype(jnp.int32)
    dst = edge_index[1].astype(jnp.int32)
    partials = _sc_edges(h2, src, typ, dst)
    return _add_call(partials.reshape(2, NPAD, D))
